# baseline (speedup 1.0000x reference)
import functools

import jax
import jax.numpy as jnp
from jax import lax
from jax.experimental import pallas as pl
from jax.experimental.pallas import tpu as pltpu
from jax.experimental.pallas import tpu_sc as plsc

B = 16384
D = 32
NC, NS = 2, 16
NW = NC * NS
BPW = B // NW
CH = 128
NCH = BPW // CH

@functools.cache
def _make_sc_gather():
    mesh = plsc.VectorSubcoreMesh(
        core_axis_name="c", subcore_axis_name="s", num_cores=NC, num_subcores=NS
    )

    @functools.partial(
        pl.kernel,
        out_type=[
            jax.ShapeDtypeStruct((B, D), jnp.float32),
            jax.ShapeDtypeStruct((B, D), jnp.float32),
        ],
        mesh=mesh,
        scratch_types=[
            pltpu.VMEM((NCH, CH), jnp.int32),
            pltpu.VMEM((NCH, CH), jnp.int32),
            pltpu.VMEM((NCH, CH, D), jnp.float32),
            pltpu.VMEM((NCH, CH, D), jnp.float32),
            pltpu.SemaphoreType.DMA,
        ],
        compiler_params=pltpu.CompilerParams(use_tc_tiling_on_sc=False),
    )
    def sc_gather(user_hbm, item_hbm, ut_hbm, it_hbm, u_out, i_out,
                  uidx, iidx, urows, irows, sem):
        wid = lax.axis_index("s") * NC + lax.axis_index("c")
        base = wid * BPW
        pltpu.sync_copy(user_hbm.at[wid], uidx)
        pltpu.sync_copy(item_hbm.at[wid], iidx)
        copies = []
        for j in range(NCH):
            copies.append(pltpu.async_copy(ut_hbm.at[uidx.at[j]], urows.at[j], sem))
            copies.append(pltpu.async_copy(it_hbm.at[iidx.at[j]], irows.at[j], sem))
        for c in copies:
            c.wait()
        for j in range(NCH):
            pltpu.sync_copy(urows.at[j], u_out.at[pl.ds(base + j * CH, CH)])
            pltpu.sync_copy(irows.at[j], i_out.at[pl.ds(base + j * CH, CH)])

    return sc_gather


BLK = 2048


def _mlp_body(u_ref, i_ref, w1u_ref, w1i_ref, b1_ref, w2_ref, b2_ref,
              w3_ref, b3_ref, o_ref):
    h = jnp.dot(u_ref[...], w1u_ref[...], preferred_element_type=jnp.float32)
    h = h + jnp.dot(i_ref[...], w1i_ref[...], preferred_element_type=jnp.float32)
    h = jnp.maximum(h + b1_ref[...], 0.0)
    h = jnp.dot(h, w2_ref[...], preferred_element_type=jnp.float32) + b2_ref[...]
    h = jnp.maximum(h, 0.0)
    z = jnp.dot(h, w3_ref[...], preferred_element_type=jnp.float32) + b3_ref[...]
    o_ref[...] = jax.nn.sigmoid(z)


def _mlp(u_g, i_g, w1u, w1i, b1, w2, b2, w3, b3):
    grid = (B // BLK,)
    full = lambda m: (0, 0)
    return pl.pallas_call(
        _mlp_body,
        grid=grid,
        in_specs=[
            pl.BlockSpec((BLK, D), lambda m: (m, 0)),
            pl.BlockSpec((BLK, D), lambda m: (m, 0)),
            pl.BlockSpec(w1u.shape, full),
            pl.BlockSpec(w1i.shape, full),
            pl.BlockSpec(b1.shape, full),
            pl.BlockSpec(w2.shape, full),
            pl.BlockSpec(b2.shape, full),
            pl.BlockSpec(w3.shape, full),
            pl.BlockSpec(b3.shape, full),
        ],
        out_specs=pl.BlockSpec((BLK, 1), lambda m: (m, 0)),
        out_shape=jax.ShapeDtypeStruct((B, 1), jnp.float32),
        compiler_params=pltpu.CompilerParams(
            dimension_semantics=("arbitrary",),
        ),
    )(u_g, i_g, w1u, w1i, b1, w2, b2, w3, b3)


def kernel(user, item, user_table, item_table, W1, b1, W2, b2, W3, b3):
    u3 = user.astype(jnp.int32).reshape(NW, NCH, CH)
    i3 = item.astype(jnp.int32).reshape(NW, NCH, CH)
    u_g, i_g = _make_sc_gather()(u3, i3, user_table, item_table)
    return _mlp(
        u_g, i_g,
        W1[:D], W1[D:],
        b1.reshape(1, -1), W2, b2.reshape(1, -1), W3, b3.reshape(1, -1),
    )
